# Initial kernel scaffold; baseline (speedup 1.0000x reference)
import jax
import jax.numpy as jnp
from jax.experimental import pallas as pl


def kernel(cls_score, label, epoch):
    raise NotImplementedError("write your pallas kernel here")


# speedup vs baseline: 7.6265x; 7.6265x over previous
import math

import jax
import jax.numpy as jnp
from jax import lax
from jax.experimental import pallas as pl
from jax.experimental.pallas import tpu as pltpu

_B = 16384
_C = 1000
_N = _B * _C
_ROWS = 512
_GRID = _B // _ROWS
_NB = 16

_CLEAN = {1: 0.9, 2: 0.8, 3: 0.7, 4: 0.6, 5: 0.5, 6: 0.4, 7: 0.3, 8: 0.2}
_K_TABLE = [0] + [math.ceil(_B * _C * (1 - _CLEAN[e])) for e in range(1, 9)]


def _bits_delta(x, y):
    y_c = jnp.maximum(y, 0)
    yf = y_c.astype(jnp.float32)
    sp = jnp.log1p(jnp.exp(-jnp.abs(x)))
    relu = jnp.maximum(x, 0.0)
    loss = relu - x * yf + sp
    u = jnp.where(y == 0, loss, 0.0)
    ubits = lax.bitcast_convert_type(u, jnp.int32)
    delta = x * (1.0 - 2.0 * yf)
    return ubits, delta, relu, sp, yf


def _hist_accum(c_ref, s_ref, chunk, delta, first):
    @pl.when(first)
    def _():
        c_ref[...] = jnp.zeros_like(c_ref)
        s_ref[...] = jnp.zeros_like(s_ref)

    cnts = []
    sums = []
    for j in range(_NB):
        m = chunk == j
        cnts.append(jnp.sum(m.astype(jnp.int32), axis=0))
        sums.append(jnp.sum(jnp.where(m, delta, 0.0), axis=0))
    c_ref[...] += jnp.stack(cnts)
    s_ref[...] += jnp.stack(sums)


def _pass0_kernel(x_ref, y_ref, c_ref, s_ref, corr_ref):
    i = pl.program_id(0)
    x = x_ref[...]
    y = y_ref[...]
    ubits, delta, relu, sp, yf = _bits_delta(x, y)
    corrected = relu - x * (1.0 - yf) + sp
    chunk = lax.shift_right_logical(ubits, 28)
    _hist_accum(c_ref, s_ref, chunk, delta, i == 0)

    @pl.when(i == 0)
    def _():
        corr_ref[...] = jnp.zeros_like(corr_ref)

    corr_ref[...] += jnp.sum(corrected, axis=0, keepdims=True)


def _passp_kernel(base_ref, x_ref, y_ref, c_ref, s_ref, *, shift):
    i = pl.program_id(0)
    base = base_ref[0, 0]
    x = x_ref[...]
    y = y_ref[...]
    ubits, delta, _, _, _ = _bits_delta(x, y)
    psh = shift + 4
    match = lax.shift_right_logical(ubits, psh) == lax.shift_right_logical(base, psh)
    chunk = jnp.where(match, lax.shift_right_logical(ubits, shift) & (_NB - 1), _NB)
    _hist_accum(c_ref, s_ref, chunk, delta, i == 0)


def _block_specs():
    in_spec = pl.BlockSpec((_ROWS, _C), lambda i: (i, 0))
    acc_spec = pl.BlockSpec((_NB, _C), lambda i: (0, 0))
    return in_spec, acc_spec


def kernel(cls_score, label, epoch):
    x = cls_score
    y = label
    in_spec, acc_spec = _block_specs()
    corr_spec = pl.BlockSpec((1, _C), lambda i: (0, 0))

    c0, s0, corr = pl.pallas_call(
        _pass0_kernel,
        grid=(_GRID,),
        in_specs=[in_spec, in_spec],
        out_specs=[acc_spec, acc_spec, corr_spec],
        out_shape=[
            jax.ShapeDtypeStruct((_NB, _C), jnp.int32),
            jax.ShapeDtypeStruct((_NB, _C), jnp.float32),
            jax.ShapeDtypeStruct((1, _C), jnp.float32),
        ],
    )(x, y)

    idx = jnp.clip(jnp.asarray(epoch, jnp.int32), 0, 8)
    k = jnp.asarray(_K_TABLE, jnp.int32)[idx]

    def select(c2d, s2d, base, prev_ge, acc, shift):
        c = jnp.sum(c2d[:_NB], axis=1)
        s = jnp.sum(s2d[:_NB], axis=1)
        above = prev_ge - jnp.sum(c)
        cum = above + jnp.cumsum(c[::-1])[::-1]
        jstar = jnp.sum((cum[1:] >= k).astype(jnp.int32))
        acc = acc + jnp.sum(jnp.where(jnp.arange(_NB) < jstar, s, 0.0))
        prev_ge = cum[jstar]
        base = base | (jstar << shift)
        return base, prev_ge, acc

    base = jnp.int32(0)
    prev_ge = jnp.int32(_N)
    acc = jnp.float32(0.0)
    base, prev_ge, acc = select(c0, s0, base, prev_ge, acc, 28)

    for shift in (24, 20, 16, 12, 8, 4, 0):
        cp, sp_ = pl.pallas_call(
            lambda b, xr, yr, cr, sr, _s=shift: _passp_kernel(b, xr, yr, cr, sr, shift=_s),
            grid=(_GRID,),
            in_specs=[
                pl.BlockSpec(memory_space=pltpu.SMEM),
                in_spec,
                in_spec,
            ],
            out_specs=[acc_spec, acc_spec],
            out_shape=[
                jax.ShapeDtypeStruct((_NB, _C), jnp.int32),
                jax.ShapeDtypeStruct((_NB, _C), jnp.float32),
            ],
        )(base.reshape(1, 1), x, y)
        base, prev_ge, acc = select(cp, sp_, base, prev_ge, acc, shift)

    total = jnp.sum(corr) + acc
    return total / jnp.float32(_N)


# speedup vs baseline: 14.5085x; 1.9024x over previous
import functools
import math

import jax
import jax.numpy as jnp
from jax import lax
from jax.experimental import pallas as pl
from jax.experimental.pallas import tpu as pltpu
from jax.experimental.pallas import tpu_sc as plsc

_B = 16384
_C = 1000
_CP = 1024
_N = _B * _C
_NP = _B * _CP
_ROWS = 512
_GRID = _B // _ROWS

_NCORES = 2
_NSUB = 16
_NTILES = _NCORES * _NSUB
_TROWS = _B // _NTILES
_CHUNK_R = 16
_VECS = _CHUNK_R * (_CP // 16)

_LEVELS = ((21, 11), (10, 11), (0, 10))

_CLEAN = {1: 0.9, 2: 0.8, 3: 0.7, 4: 0.6, 5: 0.5, 6: 0.4, 7: 0.3, 8: 0.2}
_K_TABLE = [0] + [math.ceil(_B * _C * (1 - _CLEAN[e])) for e in range(1, 9)]


def _prep_kernel(x_ref, y_ref, ub_ref, dl_ref, corr_ref):
    i = pl.program_id(0)
    x = x_ref[...]
    y = y_ref[...]
    y_c = jnp.maximum(y, 0)
    yf = y_c.astype(jnp.float32)
    sp = jnp.log1p(jnp.exp(-jnp.abs(x)))
    relu = jnp.maximum(x, 0.0)
    loss = relu - x * yf + sp
    corrected = relu - x * (1.0 - yf) + sp
    u = jnp.where(y == 0, loss, 0.0)
    ubits = lax.bitcast_convert_type(u, jnp.int32)
    delta = x * (1.0 - 2.0 * yf)
    zi = jnp.zeros((_ROWS, _CP - _C), jnp.int32)
    zf = jnp.zeros((_ROWS, _CP - _C), jnp.float32)
    ub_ref[...] = jnp.concatenate([ubits, zi], axis=1)
    dl_ref[...] = jnp.concatenate([delta, zf], axis=1)

    @pl.when(i == 0)
    def _():
        corr_ref[...] = jnp.zeros_like(corr_ref)

    corr_ref[...] += jnp.sum(corrected, axis=0, keepdims=True)


def _sc_hist_body(shift, nbits, use_mask,
                  ub_hbm, dl_hbm, base_hbm, cnt_out, sum_out,
                  ub_v, dl_v, hc_v, hs_v, base_v):
    nb = 1 << nbits
    cid = lax.axis_index("c")
    sid = lax.axis_index("s")
    wid = sid * _NCORES + cid
    row0 = wid * _TROWS

    def zero_body(i, _):
        hc_v[pl.ds(i * 16, 16)] = jnp.zeros((16,), jnp.int32)
        hs_v[pl.ds(i * 16, 16)] = jnp.zeros((16,), jnp.float32)
        return 0

    lax.fori_loop(0, nb // 16, zero_body, 0)

    pltpu.sync_copy(base_hbm, base_v)
    bvec = base_v[...]
    ones = jnp.ones((16,), jnp.int32)

    def chunk_body(g, _):
        r = row0 + g * _CHUNK_R
        pltpu.sync_copy(ub_hbm.at[pl.ds(r, _CHUNK_R)], ub_v)
        pltpu.sync_copy(dl_hbm.at[pl.ds(r, _CHUNK_R)], dl_v)

        def vec_body(i, _):
            rr = i >> 6
            cc = (i & 63) * 16
            ub = ub_v[rr, pl.ds(cc, 16)]
            dl = dl_v[rr, pl.ds(cc, 16)]
            bin_ = lax.shift_right_logical(ub, shift) & (nb - 1)
            if use_mask:
                m = lax.shift_right_logical(ub, shift + nbits) == bvec
                plsc.addupdate_scatter(hc_v, [bin_], ones, mask=m)
                plsc.addupdate_scatter(hs_v, [bin_], dl, mask=m)
            else:
                plsc.addupdate_scatter(hc_v, [bin_], ones)
                plsc.addupdate_scatter(hs_v, [bin_], dl)
            return 0

        lax.fori_loop(0, _VECS, vec_body, 0)
        return 0

    lax.fori_loop(0, _TROWS // _CHUNK_R, chunk_body, 0)

    pltpu.sync_copy(hc_v, cnt_out.at[wid])
    pltpu.sync_copy(hs_v, sum_out.at[wid])


def _make_sc_pass(shift, nbits, use_mask):
    nb = 1 << nbits
    mesh = plsc.VectorSubcoreMesh(core_axis_name="c", subcore_axis_name="s")
    return pl.kernel(
        functools.partial(_sc_hist_body, shift, nbits, use_mask),
        out_type=[
            jax.ShapeDtypeStruct((_NTILES, nb), jnp.int32),
            jax.ShapeDtypeStruct((_NTILES, nb), jnp.float32),
        ],
        mesh=mesh,
        compiler_params=pltpu.CompilerParams(needs_layout_passes=False),
        scratch_types=[
            pltpu.VMEM((_CHUNK_R, _CP), jnp.int32),
            pltpu.VMEM((_CHUNK_R, _CP), jnp.float32),
            pltpu.VMEM((nb,), jnp.int32),
            pltpu.VMEM((nb,), jnp.float32),
            pltpu.VMEM((16,), jnp.int32),
        ],
        name=f"sc_hist_{shift}_{nbits}",
    )


def kernel(cls_score, label, epoch):
    in_spec = pl.BlockSpec((_ROWS, _C), lambda i: (i, 0))
    out_spec = pl.BlockSpec((_ROWS, _CP), lambda i: (i, 0))
    corr_spec = pl.BlockSpec((1, _C), lambda i: (0, 0))

    ubits, delta, corr = pl.pallas_call(
        _prep_kernel,
        grid=(_GRID,),
        in_specs=[in_spec, in_spec],
        out_specs=[out_spec, out_spec, corr_spec],
        out_shape=[
            jax.ShapeDtypeStruct((_B, _CP), jnp.int32),
            jax.ShapeDtypeStruct((_B, _CP), jnp.float32),
            jax.ShapeDtypeStruct((1, _C), jnp.float32),
        ],
    )(cls_score, label)

    idx = jnp.clip(jnp.asarray(epoch, jnp.int32), 0, 8)
    k = jnp.asarray(_K_TABLE, jnp.int32)[idx]

    base = jnp.int32(0)
    prev_ge = jnp.int32(_NP)
    acc = jnp.float32(0.0)

    for li, (shift, nbits) in enumerate(_LEVELS):
        nb = 1 << nbits
        bshift = lax.shift_right_logical(base, shift + nbits)
        bvec = jnp.broadcast_to(bshift, (16,)).astype(jnp.int32)
        cnt2d, sum2d = _make_sc_pass(shift, nbits, li > 0)(ubits, delta, bvec)
        c = jnp.sum(cnt2d, axis=0)
        s = jnp.sum(sum2d, axis=0)
        above = prev_ge - jnp.sum(c)
        cum = above + jnp.cumsum(c[::-1])[::-1]
        jstar = jnp.sum((cum[1:] >= k).astype(jnp.int32))
        acc = acc + jnp.sum(jnp.where(jnp.arange(nb) < jstar, s, 0.0))
        prev_ge = cum[jstar]
        base = base | (jstar << shift)

    total = jnp.sum(corr) + acc
    return total / jnp.float32(_N)


# speedup vs baseline: 16.8302x; 1.1600x over previous
import functools
import math

import jax
import jax.numpy as jnp
from jax import lax
from jax.experimental import pallas as pl
from jax.experimental.pallas import tpu as pltpu
from jax.experimental.pallas import tpu_sc as plsc

_B = 16384
_C = 1000
_CP = 1024
_N = _B * _C
_NP = _B * _CP
_ROWS = 512
_GRID = _B // _ROWS

_NCORES = 2
_NSUB = 16
_NTILES = _NCORES * _NSUB
_TROWS = _B // _NTILES
_CHUNK_R = 16
_VECS = _CHUNK_R * (_CP // 16)

_LEVELS = ((21, 11), (10, 11), (0, 10))

_CLEAN = {1: 0.9, 2: 0.8, 3: 0.7, 4: 0.6, 5: 0.5, 6: 0.4, 7: 0.3, 8: 0.2}
_K_TABLE = [0] + [math.ceil(_B * _C * (1 - _CLEAN[e])) for e in range(1, 9)]


def _prep_kernel(x_ref, y_ref, ub_ref, dl_ref, corr_ref):
    i = pl.program_id(0)
    x = x_ref[...]
    y = y_ref[...]
    y_c = jnp.maximum(y, 0)
    yf = y_c.astype(jnp.float32)
    sp = jnp.log1p(jnp.exp(-jnp.abs(x)))
    relu = jnp.maximum(x, 0.0)
    loss = relu - x * yf + sp
    corrected = relu - x * (1.0 - yf) + sp
    u = jnp.where(y == 0, loss, 0.0)
    ubits = lax.bitcast_convert_type(u, jnp.int32)
    delta = x * (1.0 - 2.0 * yf)
    zi = jnp.zeros((_ROWS, _CP - _C), jnp.int32)
    zf = jnp.zeros((_ROWS, _CP - _C), jnp.float32)
    ub_ref[...] = jnp.concatenate([ubits, zi], axis=1)
    dl_ref[...] = jnp.concatenate([delta, zf], axis=1)

    @pl.when(i == 0)
    def _():
        corr_ref[...] = jnp.zeros_like(corr_ref)

    corr_ref[...] += jnp.sum(corrected, axis=0, keepdims=True)


def _sc_hist_body(shift, nbits, use_mask,
                  ub_hbm, dl_hbm, base_hbm, cnt_out, sum_out,
                  ub_v, dl_v, hc_v, hs_v, base_v):
    nb = 1 << nbits
    cid = lax.axis_index("c")
    sid = lax.axis_index("s")
    wid = sid * _NCORES + cid
    row0 = wid * _TROWS

    def zero_body(i, _):
        for u in range(4):
            hc_v[pl.ds((i * 4 + u) * 16, 16)] = jnp.zeros((16,), jnp.int32)
            hs_v[pl.ds((i * 4 + u) * 16, 16)] = jnp.zeros((16,), jnp.float32)
        return 0

    lax.fori_loop(0, nb // 4, zero_body, 0)

    pltpu.sync_copy(base_hbm, base_v)
    bvec = base_v[...]
    ones = jnp.ones((16,), jnp.int32)
    lane = jnp.arange(16, dtype=jnp.int32)

    def chunk_body(g, _):
        r = row0 + g * _CHUNK_R
        pltpu.sync_copy(ub_hbm.at[pl.ds(r, _CHUNK_R)], ub_v)
        pltpu.sync_copy(dl_hbm.at[pl.ds(r, _CHUNK_R)], dl_v)

        def vec_body(i, _):
            for u in range(4):
                v = i * 4 + u
                rr = v >> 6
                cc = (v & 63) * 16
                ub = ub_v[rr, pl.ds(cc, 16)]
                dl = dl_v[rr, pl.ds(cc, 16)]
                bin_ = lax.shift_right_logical(ub, shift) & (nb - 1)
                slot = (bin_ << 4) | lane
                if use_mask:
                    m = lax.shift_right_logical(ub, shift + nbits) == bvec
                    plsc.addupdate_scatter(hc_v, [slot], ones, mask=m)
                    plsc.addupdate_scatter(hs_v, [slot], dl, mask=m)
                else:
                    plsc.addupdate_scatter(hc_v, [slot], ones)
                    plsc.addupdate_scatter(hs_v, [slot], dl)
            return 0

        lax.fori_loop(0, _VECS // 4, vec_body, 0)
        return 0

    lax.fori_loop(0, _TROWS // _CHUNK_R, chunk_body, 0)

    pltpu.sync_copy(hc_v, cnt_out.at[wid])
    pltpu.sync_copy(hs_v, sum_out.at[wid])


def _make_sc_pass(shift, nbits, use_mask):
    nb = 1 << nbits
    mesh = plsc.VectorSubcoreMesh(core_axis_name="c", subcore_axis_name="s")
    return pl.kernel(
        functools.partial(_sc_hist_body, shift, nbits, use_mask),
        out_type=[
            jax.ShapeDtypeStruct((_NTILES, nb * 16), jnp.int32),
            jax.ShapeDtypeStruct((_NTILES, nb * 16), jnp.float32),
        ],
        mesh=mesh,
        compiler_params=pltpu.CompilerParams(needs_layout_passes=False),
        scratch_types=[
            pltpu.VMEM((_CHUNK_R, _CP), jnp.int32),
            pltpu.VMEM((_CHUNK_R, _CP), jnp.float32),
            pltpu.VMEM((nb * 16,), jnp.int32),
            pltpu.VMEM((nb * 16,), jnp.float32),
            pltpu.VMEM((16,), jnp.int32),
        ],
        name=f"sc_hist_{shift}_{nbits}",
    )


def kernel(cls_score, label, epoch):
    in_spec = pl.BlockSpec((_ROWS, _C), lambda i: (i, 0))
    out_spec = pl.BlockSpec((_ROWS, _CP), lambda i: (i, 0))
    corr_spec = pl.BlockSpec((1, _C), lambda i: (0, 0))

    ubits, delta, corr = pl.pallas_call(
        _prep_kernel,
        grid=(_GRID,),
        in_specs=[in_spec, in_spec],
        out_specs=[out_spec, out_spec, corr_spec],
        out_shape=[
            jax.ShapeDtypeStruct((_B, _CP), jnp.int32),
            jax.ShapeDtypeStruct((_B, _CP), jnp.float32),
            jax.ShapeDtypeStruct((1, _C), jnp.float32),
        ],
    )(cls_score, label)

    idx = jnp.clip(jnp.asarray(epoch, jnp.int32), 0, 8)
    k = jnp.asarray(_K_TABLE, jnp.int32)[idx]

    base = jnp.int32(0)
    prev_ge = jnp.int32(_NP)
    acc = jnp.float32(0.0)

    for li, (shift, nbits) in enumerate(_LEVELS):
        nb = 1 << nbits
        bshift = lax.shift_right_logical(base, shift + nbits)
        bvec = jnp.broadcast_to(bshift, (16,)).astype(jnp.int32)
        cnt2d, sum2d = _make_sc_pass(shift, nbits, li > 0)(ubits, delta, bvec)
        c = jnp.sum(cnt2d.reshape(_NTILES, nb, 16), axis=(0, 2))
        s = jnp.sum(sum2d.reshape(_NTILES, nb, 16), axis=(0, 2))
        above = prev_ge - jnp.sum(c)
        cum = above + jnp.cumsum(c[::-1])[::-1]
        jstar = jnp.sum((cum[1:] >= k).astype(jnp.int32))
        acc = acc + jnp.sum(jnp.where(jnp.arange(nb) < jstar, s, 0.0))
        prev_ge = cum[jstar]
        base = base | (jstar << shift)

    total = jnp.sum(corr) + acc
    return total / jnp.float32(_N)


# speedup vs baseline: 22.8862x; 1.3598x over previous
import functools
import math

import jax
import jax.numpy as jnp
from jax import lax
from jax.experimental import pallas as pl
from jax.experimental.pallas import tpu as pltpu
from jax.experimental.pallas import tpu_sc as plsc

_B = 16384
_C = 1000
_CP = 1024
_N = _B * _C
_NP = _B * _CP
_ROWS = 512
_GRID = _B // _ROWS

_NCORES = 2
_NSUB = 16
_NTILES = _NCORES * _NSUB
_TROWS = _B // _NTILES
_CHUNK_R = 16
_VECS = _CHUNK_R * (_CP // 16)
_NB = 1 << 16

_CLEAN = {1: 0.9, 2: 0.8, 3: 0.7, 4: 0.6, 5: 0.5, 6: 0.4, 7: 0.3, 8: 0.2}
_K_TABLE = [0] + [math.ceil(_B * _C * (1 - _CLEAN[e])) for e in range(1, 9)]

_SC_PARAMS = pltpu.CompilerParams(needs_layout_passes=False)
_INF_BITS = 0x7F800000


def _prep_kernel(x_ref, y_ref, ub_ref, dl_ref, corr_ref):
    i = pl.program_id(0)
    x = x_ref[...]
    y = y_ref[...]
    y_c = jnp.maximum(y, 0)
    yf = y_c.astype(jnp.float32)
    sp = jnp.log1p(jnp.exp(-jnp.abs(x)))
    relu = jnp.maximum(x, 0.0)
    loss = relu - x * yf + sp
    corrected = relu - x * (1.0 - yf) + sp
    u = jnp.where(y == 0, loss, 0.0)
    ubits = lax.bitcast_convert_type(u, jnp.int32)
    delta = x * (1.0 - 2.0 * yf)
    zi = jnp.zeros((_ROWS, _CP - _C), jnp.int32)
    zf = jnp.zeros((_ROWS, _CP - _C), jnp.float32)
    ub_ref[...] = jnp.concatenate([ubits, zi], axis=1)
    dl_ref[...] = jnp.concatenate([delta, zf], axis=1)

    @pl.when(i == 0)
    def _():
        corr_ref[...] = jnp.zeros_like(corr_ref)

    corr_ref[...] += jnp.sum(corrected, axis=0, keepdims=True)


def _tile_id():
    return lax.axis_index("s") * _NCORES + lax.axis_index("c")


def _sc_cnt_body(lo_level, ub_hbm, base_hbm, cnt_out, ub_v, hc_v, base_v):
    wid = _tile_id()
    row0 = wid * _TROWS

    def zero_body(i, _):
        for u in range(4):
            hc_v[pl.ds((i * 4 + u) * 16, 16)] = jnp.zeros((16,), jnp.int32)
        return 0

    lax.fori_loop(0, _NB // 64, zero_body, 0)

    pltpu.sync_copy(base_hbm, base_v)
    bvec = base_v[...]
    ones = jnp.ones((16,), jnp.int32)
    zeros16 = jnp.zeros((16,), jnp.int32)

    def chunk_body(g, _):
        r = row0 + g * _CHUNK_R
        pltpu.sync_copy(ub_hbm.at[pl.ds(r, _CHUNK_R)], ub_v)

        def vec_body(i, _):
            for u in range(4):
                v = i * 4 + u
                ub = ub_v[v >> 6, pl.ds((v & 63) * 16, 16)]
                hi = lax.shift_right_logical(ub, 16)
                if lo_level:
                    m = hi == bvec
                    bin_ = ub & (_NB - 1)
                else:
                    m = ub != zeros16
                    bin_ = hi
                plsc.addupdate_scatter(hc_v, [bin_], ones, mask=m)
            return 0

        lax.fori_loop(0, _VECS // 4, vec_body, 0)
        return 0

    lax.fori_loop(0, _TROWS // _CHUNK_R, chunk_body, 0)
    pltpu.sync_copy(hc_v, cnt_out.at[wid])


def _sc_sum_body(ub_hbm, dl_hbm, thr_hbm, sum_out, ub_v, dl_v, thr_v, acc_v):
    wid = _tile_id()
    row0 = wid * _TROWS
    pltpu.sync_copy(thr_hbm, thr_v)
    tvec = thr_v[...]
    zf = jnp.zeros((16,), jnp.float32)

    def chunk_body(g, acc):
        r = row0 + g * _CHUNK_R
        pltpu.sync_copy(ub_hbm.at[pl.ds(r, _CHUNK_R)], ub_v)
        pltpu.sync_copy(dl_hbm.at[pl.ds(r, _CHUNK_R)], dl_v)

        def vec_body(i, a):
            for u in range(4):
                v = i * 4 + u
                ub = ub_v[v >> 6, pl.ds((v & 63) * 16, 16)]
                dl = dl_v[v >> 6, pl.ds((v & 63) * 16, 16)]
                a = a + jnp.where(ub < tvec, dl, zf)
            return a

        return lax.fori_loop(0, _VECS // 4, vec_body, acc)

    acc = lax.fori_loop(0, _TROWS // _CHUNK_R, chunk_body, zf)
    acc_v[...] = acc
    pltpu.sync_copy(acc_v, sum_out.at[wid])


_SC_MESH = plsc.VectorSubcoreMesh(core_axis_name="c", subcore_axis_name="s")


def _make_cnt_pass(lo_level):
    return pl.kernel(
        functools.partial(_sc_cnt_body, lo_level),
        out_type=jax.ShapeDtypeStruct((_NTILES, _NB), jnp.int32),
        mesh=_SC_MESH,
        compiler_params=_SC_PARAMS,
        scratch_types=[
            pltpu.VMEM((_CHUNK_R, _CP), jnp.int32),
            pltpu.VMEM((_NB,), jnp.int32),
            pltpu.VMEM((16,), jnp.int32),
        ],
        name=f"sc_cnt_{int(lo_level)}",
    )


_SUM_PASS = pl.kernel(
    _sc_sum_body,
    out_type=jax.ShapeDtypeStruct((_NTILES, 16), jnp.float32),
    mesh=_SC_MESH,
    compiler_params=_SC_PARAMS,
    scratch_types=[
        pltpu.VMEM((_CHUNK_R, _CP), jnp.int32),
        pltpu.VMEM((_CHUNK_R, _CP), jnp.float32),
        pltpu.VMEM((16,), jnp.int32),
        pltpu.VMEM((16,), jnp.float32),
    ],
    name="sc_sum",
)


def _select(c, prev_ge, k):
    above = prev_ge - jnp.sum(c)
    cum = above + jnp.cumsum(c[::-1])[::-1]
    jstar = jnp.sum((cum[1:] >= k).astype(jnp.int32))
    return jstar, cum[jstar]


def kernel(cls_score, label, epoch):
    in_spec = pl.BlockSpec((_ROWS, _C), lambda i: (i, 0))
    out_spec = pl.BlockSpec((_ROWS, _CP), lambda i: (i, 0))
    corr_spec = pl.BlockSpec((1, _C), lambda i: (0, 0))

    ubits, delta, corr = pl.pallas_call(
        _prep_kernel,
        grid=(_GRID,),
        in_specs=[in_spec, in_spec],
        out_specs=[out_spec, out_spec, corr_spec],
        out_shape=[
            jax.ShapeDtypeStruct((_B, _CP), jnp.int32),
            jax.ShapeDtypeStruct((_B, _CP), jnp.float32),
            jax.ShapeDtypeStruct((1, _C), jnp.float32),
        ],
    )(cls_score, label)

    idx = jnp.clip(jnp.asarray(epoch, jnp.int32), 0, 8)
    k = jnp.asarray(_K_TABLE, jnp.int32)[idx]

    zeros16 = jnp.zeros((16,), jnp.int32)
    cnt_hi = _make_cnt_pass(False)(ubits, zeros16)
    c_hi = jnp.sum(cnt_hi, axis=0)
    c_hi = c_hi.at[0].add(_NP - jnp.sum(c_hi))
    h_star, ge_hi = _select(c_hi, jnp.int32(_NP), k)

    cnt_lo = _make_cnt_pass(True)(ubits, jnp.broadcast_to(h_star, (16,)))
    c_lo = jnp.sum(cnt_lo, axis=0)
    l_star, _ = _select(c_lo, ge_hi, k)

    t_bits = (h_star << 16) | l_star
    t_bits = jnp.where(t_bits < 0, jnp.int32(_INF_BITS), t_bits)
    sums = _SUM_PASS(ubits, delta, jnp.broadcast_to(t_bits, (16,)))
    acc = jnp.sum(sums)

    total = jnp.sum(corr) + acc
    return total / jnp.float32(_N)
